# baseline (speedup 1.0000x reference)
import dataclasses

import jax
import jax.numpy as jnp
from jax import lax
from jax.experimental import pallas as pl
from jax.experimental.pallas import tpu as pltpu
from jax.experimental.pallas import tpu_sc as plsc

B = 16384
L = 50
D = 32
H = D // 2
V = 1000000

NUM_CORES = 2
NUM_SUBCORES = 16
NW = NUM_CORES * NUM_SUBCORES
RPW = B // NW
CHUNK = 64
NCHUNK = RPW // CHUNK
NIDX = CHUNK * L
XFER = 128
NXFER = NIDX // XFER
IDX_PAD = NIDX + 64

SLAB = 320
NSLAB = V // SLAB
SLAB_X = SLAB // 4
SLAB_PW = -(-NSLAB // NW)
XROWS = V // 4


def _tree_sum(xs):
    while len(xs) > 1:
        ys = [xs[i] + xs[i + 1] for i in range(0, len(xs) - 1, 2)]
        if len(xs) % 2:
            ys.append(xs[-1])
        xs = ys
    return xs[0]


def _detile_body(t_hbm, x_hbm, in0, in1, out0, out1,
                 semi0, semi1, semo0, semo1):
    wid = lax.axis_index("s") * NUM_CORES + lax.axis_index("c")
    ins = (in0, in1)
    outs = (out0, out1)
    semis = (semi0, semi1)
    semos = (semo0, semo1)

    def slab_id(k):
        return wid + k * NW

    def fire_in(k, par):
        pltpu.async_copy(t_hbm.at[pl.ds(slab_id(k) * SLAB, SLAB)],
                         ins[par], semis[par])

    def remap(k, par):
        ib, ob = ins[par], outs[par]
        pltpu.make_async_copy(t_hbm.at[pl.ds(0, SLAB)], ib,
                              semis[par]).wait()

        @pl.loop(0, SLAB_X, step=4)
        def _row(rr):
            vals = []
            for r in range(4):
                for cc in range(8):
                    vals.append(ib[(rr + r) * 4 + cc // 2,
                                   pl.ds((cc % 2) * 16, 16)])
            for r in range(4):
                for cc in range(8):
                    ob[rr + r, pl.ds(cc * 16, 16)] = vals[r * 8 + cc]

        pltpu.async_copy(ob, x_hbm.at[pl.ds(slab_id(k) * SLAB_X, SLAB_X)],
                         semos[par])

    fire_in(0, 0)

    @pl.loop(0, SLAB_PW + 1, step=2)
    def _steps(k):
        for par in range(2):
            cur = k + par

            @pl.when(cur < SLAB_PW)
            def _():
                @pl.when(jnp.logical_and(cur + 1 < SLAB_PW,
                                         slab_id(cur + 1) < NSLAB))
                def _():
                    fire_in(cur + 1, (par + 1) % 2)

                @pl.when(slab_id(cur) < NSLAB)
                def _():
                    @pl.when(cur >= 2)
                    def _():
                        pltpu.make_async_copy(x_hbm.at[pl.ds(0, SLAB_X)],
                                              outs[par].at[pl.ds(0, SLAB_X)],
                                              semos[par]).wait()

                    remap(cur, par)

    @pl.when(slab_id(0) < NSLAB)
    def _d0():
        pltpu.make_async_copy(x_hbm.at[pl.ds(0, SLAB_X)],
                              outs[0].at[pl.ds(0, SLAB_X)], semos[0]).wait()

    @pl.when(slab_id(1) < NSLAB)
    def _d1():
        pltpu.make_async_copy(x_hbm.at[pl.ds(0, SLAB_X)],
                              outs[1].at[pl.ds(0, SLAB_X)], semos[1]).wait()


def _embed_mean_body(words_hbm, table_hbm, out_hbm, idx_v, rows_v, out_v,
                     t0_v, sem):
    wid = lax.axis_index("s") * NUM_CORES + lax.axis_index("c")
    pltpu.sync_copy(table_hbm.at[pl.ds(0, 1)], t0_v)
    t0_lo = t0_v[0, pl.ds(0, H)]
    t0_hi = t0_v[0, pl.ds(H, H)]
    lanes = lax.iota(jnp.int32, 16)
    scale = jnp.float32(1.0 / L)

    @pl.loop(0, NCHUNK)
    def _chunk(c):
        start = wid * (RPW * L) + c * NIDX
        pltpu.sync_copy(words_hbm.at[pl.ds(start, NIDX)],
                        idx_v.at[pl.ds(0, NIDX)])
        copies = [
            pltpu.async_copy(
                table_hbm.at[idx_v.at[pl.ds(j * XFER, XFER)]],
                rows_v.at[pl.ds(j * XFER, XFER)],
                sem,
            )
            for j in range(NXFER)
        ]
        for cp in copies:
            cp.wait()

        @pl.loop(0, CHUNK)
        def _row(i):
            base = i * L
            lo = [rows_v[base + j, pl.ds(0, H)] for j in range(L)]
            hi = [rows_v[base + j, pl.ds(H, H)] for j in range(L)]
            acc_lo = _tree_sum(lo)
            acc_hi = _tree_sum(hi)
            nz = jnp.float32(0.0)
            for q in range(4):
                pos = base + q * 16 + lanes
                if (q + 1) * 16 <= L:
                    vals = plsc.load_gather(idx_v, [pos])
                    hit = vals == 0
                else:
                    live = lanes < jnp.int32(L - q * 16)
                    vals = plsc.load_gather(idx_v, [pos], mask=live)
                    hit = jnp.logical_and(vals == 0, live)
                nz = nz + jnp.sum(jnp.where(hit, jnp.float32(1.0),
                                            jnp.float32(0.0)))
            out_v[i, pl.ds(0, H)] = (acc_lo - nz * t0_lo) * scale
            out_v[i, pl.ds(H, H)] = (acc_hi - nz * t0_hi) * scale

        pltpu.sync_copy(out_v,
                        out_hbm.at[pl.ds(wid * RPW + c * CHUNK, CHUNK)])


def kernel(words, table):
    mesh = plsc.VectorSubcoreMesh(core_axis_name="c", subcore_axis_name="s")

    cp_tiled = pltpu.CompilerParams(use_tc_tiling_on_sc=True,
                                    disable_bounds_checks=True)
    cp_lin = pltpu.CompilerParams(use_tc_tiling_on_sc=False,
                                  disable_bounds_checks=True)
    if "needs_layout_passes" in pltpu.CompilerParams.__dataclass_fields__:
        cp_tiled = dataclasses.replace(cp_tiled, needs_layout_passes=False)
        cp_lin = dataclasses.replace(cp_lin, needs_layout_passes=False)

    detile = pl.kernel(
        _detile_body,
        out_type=jax.ShapeDtypeStruct((XROWS, 128), jnp.float32),
        mesh=mesh,
        scratch_types=[
            pltpu.VMEM((SLAB, D), jnp.float32),
            pltpu.VMEM((SLAB, D), jnp.float32),
            pltpu.VMEM((SLAB_X, 128), jnp.float32),
            pltpu.VMEM((SLAB_X, 128), jnp.float32),
            pltpu.SemaphoreType.DMA,
            pltpu.SemaphoreType.DMA,
            pltpu.SemaphoreType.DMA,
            pltpu.SemaphoreType.DMA,
        ],
        compiler_params=cp_tiled,
    )
    x = detile(table)
    table_lin = x.reshape(V, D)

    words_flat = words.reshape(B * L)
    embed = pl.kernel(
        _embed_mean_body,
        out_type=jax.ShapeDtypeStruct((B, D), jnp.float32),
        mesh=mesh,
        scratch_types=[
            pltpu.VMEM((IDX_PAD,), jnp.int32),
            pltpu.VMEM((NIDX, D), jnp.float32),
            pltpu.VMEM((CHUNK, D), jnp.float32),
            pltpu.VMEM((1, D), jnp.float32),
            pltpu.SemaphoreType.DMA,
        ],
        compiler_params=cp_lin,
    )
    return embed(words_flat, table_lin)


# speedup vs baseline: 1.1036x; 1.1036x over previous
import dataclasses

import jax
import jax.numpy as jnp
from jax import lax
from jax.experimental import pallas as pl
from jax.experimental.pallas import tpu as pltpu
from jax.experimental.pallas import tpu_sc as plsc

B = 16384
L = 50
D = 32
H = D // 2

NUM_CORES = 2
NUM_SUBCORES = 16
NW = NUM_CORES * NUM_SUBCORES
RPW = B // NW
CHUNK = 32
NCHUNK = RPW // CHUNK
NIDX = CHUNK * L
XFER = 64
NXFER = NIDX // XFER
IDX_PAD = NIDX + 64


def _tree_sum(xs):
    while len(xs) > 1:
        ys = [xs[i] + xs[i + 1] for i in range(0, len(xs) - 1, 2)]
        if len(xs) % 2:
            ys.append(xs[-1])
        xs = ys
    return xs[0]


def _embed_mean_body(words_hbm, table_hbm, out_hbm,
                     idx0, idx1, rows0, rows1, out0, out1, t0_v,
                     semg0, semg1, semo0, semo1):
    wid = lax.axis_index("s") * NUM_CORES + lax.axis_index("c")
    pltpu.sync_copy(table_hbm.at[pl.ds(0, 1)], t0_v)
    t0_lo = t0_v[0, pl.ds(0, H)]
    t0_hi = t0_v[0, pl.ds(H, H)]
    lanes = lax.iota(jnp.int32, 16)
    scale = jnp.float32(1.0 / L)

    idx_bufs = (idx0, idx1)
    rows_bufs = (rows0, rows1)
    out_bufs = (out0, out1)
    gsems = (semg0, semg1)
    osems = (semo0, semo1)

    def fetch(c, par):
        ib, rb = idx_bufs[par], rows_bufs[par]
        start = wid * (RPW * L) + c * NIDX
        pltpu.sync_copy(words_hbm.at[pl.ds(start, NIDX)],
                        ib.at[pl.ds(0, NIDX)])
        for j in range(NXFER):
            pltpu.async_copy(table_hbm.at[ib.at[pl.ds(j * XFER, XFER)]],
                             rb.at[pl.ds(j * XFER, XFER)], gsems[par])

    def compute(c, par):
        ib, rb, ob = idx_bufs[par], rows_bufs[par], out_bufs[par]
        pltpu.make_async_copy(table_hbm.at[pl.ds(0, NIDX)], rb,
                              gsems[par]).wait()

        @pl.loop(0, CHUNK)
        def _row(i):
            base = i * L
            lo = [rb[base + j, pl.ds(0, H)] for j in range(L)]
            hi = [rb[base + j, pl.ds(H, H)] for j in range(L)]
            acc_lo = _tree_sum(lo)
            acc_hi = _tree_sum(hi)
            nz = jnp.float32(0.0)
            for q in range(4):
                pos = base + q * 16 + lanes
                if (q + 1) * 16 <= L:
                    vals = plsc.load_gather(ib, [pos])
                    hit = vals == 0
                else:
                    live = lanes < jnp.int32(L - q * 16)
                    vals = plsc.load_gather(ib, [pos], mask=live)
                    hit = jnp.logical_and(vals == 0, live)
                nz = nz + jnp.sum(jnp.where(hit, jnp.float32(1.0),
                                            jnp.float32(0.0)))
            ob[i, pl.ds(0, H)] = (acc_lo - nz * t0_lo) * scale
            ob[i, pl.ds(H, H)] = (acc_hi - nz * t0_hi) * scale

        pltpu.async_copy(ob,
                         out_hbm.at[pl.ds(wid * RPW + c * CHUNK, CHUNK)],
                         osems[par])

    fetch(0, 0)

    @pl.loop(0, NCHUNK, step=2)
    def _steps(c):
        for par in range(2):
            cc = c + par

            @pl.when(cc + 1 < NCHUNK)
            def _():
                fetch(cc + 1, (par + 1) % 2)

            @pl.when(cc >= 2)
            def _():
                pltpu.make_async_copy(out_hbm.at[pl.ds(0, CHUNK)],
                                      out_bufs[par], osems[par]).wait()

            compute(cc, par)

    pltpu.make_async_copy(out_hbm.at[pl.ds(0, CHUNK)], out_bufs[0],
                          osems[0]).wait()
    pltpu.make_async_copy(out_hbm.at[pl.ds(0, CHUNK)], out_bufs[1],
                          osems[1]).wait()


def kernel(words, table):
    words_flat = words.reshape(B * L)
    mesh = plsc.VectorSubcoreMesh(core_axis_name="c", subcore_axis_name="s")
    cp = pltpu.CompilerParams(use_tc_tiling_on_sc=False,
                              disable_bounds_checks=True)
    if "needs_layout_passes" in pltpu.CompilerParams.__dataclass_fields__:
        cp = dataclasses.replace(cp, needs_layout_passes=False)
    f = pl.kernel(
        _embed_mean_body,
        out_type=jax.ShapeDtypeStruct((B, D), jnp.float32),
        mesh=mesh,
        scratch_types=[
            pltpu.VMEM((IDX_PAD,), jnp.int32),
            pltpu.VMEM((IDX_PAD,), jnp.int32),
            pltpu.VMEM((NIDX, D), jnp.float32),
            pltpu.VMEM((NIDX, D), jnp.float32),
            pltpu.VMEM((CHUNK, D), jnp.float32),
            pltpu.VMEM((CHUNK, D), jnp.float32),
            pltpu.VMEM((1, D), jnp.float32),
            pltpu.SemaphoreType.DMA,
            pltpu.SemaphoreType.DMA,
            pltpu.SemaphoreType.DMA,
            pltpu.SemaphoreType.DMA,
        ],
        compiler_params=cp,
    )
    return f(words_flat, table)
